# baseline (speedup 1.0000x reference)
import functools

import jax
import jax.numpy as jnp
from jax import lax
from jax.experimental import pallas as pl
from jax.experimental.pallas import tpu as pltpu
from jax.experimental.pallas import tpu_sc as plsc

B, N, D = 4, 4096, 1024
NC, NS = 2, 16
NW = NC * NS
NPW = N // NW
C = 8
NCH = NPW // C
CW = C * D

_mesh = plsc.VectorSubcoreMesh(core_axis_name="c", subcore_axis_name="s")


@functools.partial(
    pl.kernel,
    mesh=_mesh,
    out_type=jax.ShapeDtypeStruct((B * N, D), jnp.float32),
    scratch_types=(
        [pltpu.VMEM((C, D), jnp.float32)] * 2
        + [pltpu.VMEM((C, D), jnp.float32)] * (2 * B)
        + [pltpu.SemaphoreType.DMA] * 2
        + [pltpu.SemaphoreType.DMA] * (2 * B)
        + [pltpu.SemaphoreType.DMA] * (2 * B)
    ),
)
def _pos_add(x_hbm, t_hbm, o_hbm, *rest):
    tbufs = rest[0:2]
    xbufs = [rest[2 + 2 * b:4 + 2 * b] for b in range(B)]
    tsems = rest[2 + 2 * B:4 + 2 * B]
    ldsems = [rest[4 + 2 * B + 2 * b:6 + 2 * B + 2 * b] for b in range(B)]
    stsems = [rest[4 + 4 * B + 2 * b:6 + 4 * B + 2 * b] for b in range(B)]

    wid = lax.axis_index("s") * NC + lax.axis_index("c")
    nbase = wid * NPW

    def x_rows(b_, nc_):
        return pl.ds(b_ * N + nbase + nc_ * C, C)

    def t_rows(nc_):
        return pl.ds(nbase + nc_ * C, C)

    t_h = [None, None]
    t_h[0] = pltpu.async_copy(t_hbm.at[t_rows(0)], tbufs[0], tsems[0])
    ld_h = [[None, None] for _ in range(B)]
    st_h = [[None, None] for _ in range(B)]
    for b in range(B):
        ld_h[b][0] = pltpu.async_copy(
            x_hbm.at[x_rows(b, 0)], xbufs[b][0], ldsems[b][0])

    for nc in range(NCH):
        p = nc % 2
        tbuf = tbufs[p]
        t_h[p].wait()
        if nc + 1 < NCH:
            t_h[1 - p] = pltpu.async_copy(
                t_hbm.at[t_rows(nc + 1)], tbufs[1 - p], tsems[1 - p])
        for b in range(B):
            ld_h[b][p].wait()
        xcur = [xbufs[b][p] for b in range(B)]

        @plsc.parallel_loop(0, CW, step=16, unroll=4)
        def add_body(i, xcur=xcur, tbuf=tbuf):
            r = i >> 10
            c = pl.multiple_of(i & (D - 1), 16)
            sl = pl.ds(c, 16)
            tv = tbuf[r, sl]
            for b in range(B):
                plsc.addupdate(xcur[b].at[r, sl], tv)

        for b in range(B):
            st_h[b][p] = pltpu.async_copy(
                xcur[b], o_hbm.at[x_rows(b, nc)], stsems[b][p])
        if nc + 1 < NCH:
            for b in range(B):
                if st_h[b][1 - p] is not None:
                    st_h[b][1 - p].wait()
                    st_h[b][1 - p] = None
                ld_h[b][1 - p] = pltpu.async_copy(
                    x_hbm.at[x_rows(b, nc + 1)], xbufs[b][1 - p], ldsems[b][1 - p])

    for hs in st_h:
        for h in hs:
            if h is not None:
                h.wait()


def kernel(x, table):
    out = _pos_add(x.reshape(B * N, D), table)
    return out.reshape(x.shape)


# speedup vs baseline: 1.2077x; 1.2077x over previous
import functools

import jax
import jax.numpy as jnp
from jax import lax
from jax.experimental import pallas as pl
from jax.experimental.pallas import tpu as pltpu
from jax.experimental.pallas import tpu_sc as plsc

B, N, D = 4, 4096, 1024
NC, NS = 2, 16
NW = NC * NS
NPW = N // NW
C = 16
NCH = NPW // C
TOT = NCH * B
CW = C * D
NSLOT = 6
P = 3

_mesh = plsc.VectorSubcoreMesh(core_axis_name="c", subcore_axis_name="s")


@functools.partial(
    pl.kernel,
    mesh=_mesh,
    out_type=jax.ShapeDtypeStruct((B * N, D), jnp.float32),
    scratch_types=(
        [pltpu.VMEM((C, D), jnp.float32)]
        + [pltpu.VMEM((C, D), jnp.float32)] * NSLOT
        + [pltpu.SemaphoreType.DMA] * NSLOT
        + [pltpu.SemaphoreType.DMA] * NSLOT
    ),
)
def _pos_add(x_hbm, t_hbm, o_hbm, *rest):
    tbuf = rest[0]
    xbufs = rest[1:1 + NSLOT]
    ldsems = rest[1 + NSLOT:1 + 2 * NSLOT]
    stsems = rest[1 + 2 * NSLOT:1 + 3 * NSLOT]

    wid = lax.axis_index("s") * NC + lax.axis_index("c")
    nbase = wid * NPW

    def x_slice(k):
        nc_, b_ = k // B, k % B
        return pl.ds(b_ * N + nbase + nc_ * C, C)

    def t_slice(nc_):
        return pl.ds(nbase + nc_ * C, C)

    ld_h = [None] * NSLOT
    st_h = [None] * NSLOT
    for k in range(min(P, TOT)):
        ld_h[k % NSLOT] = pltpu.async_copy(
            x_hbm.at[x_slice(k)], xbufs[k % NSLOT], ldsems[k % NSLOT])

    for k in range(TOT):
        s = k % NSLOT
        nc_, b_ = k // B, k % B
        if b_ == 0:
            pltpu.sync_copy(t_hbm.at[t_slice(nc_)], tbuf)
        ld_h[s].wait()
        xb = xbufs[s]

        @plsc.parallel_loop(0, CW, step=16, unroll=8)
        def add_body(i, xb=xb, tbuf=tbuf):
            r = i >> 10
            c = pl.multiple_of(i & (D - 1), 16)
            sl = pl.ds(c, 16)
            plsc.addupdate(xb.at[r, sl], tbuf[r, sl])

        st_h[s] = pltpu.async_copy(xb, o_hbm.at[x_slice(k)], stsems[s])
        kn = k + P
        if kn < TOT:
            sn = kn % NSLOT
            if st_h[sn] is not None:
                st_h[sn].wait()
                st_h[sn] = None
            ld_h[sn] = pltpu.async_copy(x_hbm.at[x_slice(kn)], xbufs[sn], ldsems[sn])

    for h in st_h:
        if h is not None:
            h.wait()


def kernel(x, table):
    out = _pos_add(x.reshape(B * N, D), table)
    return out.reshape(x.shape)


# speedup vs baseline: 1.3437x; 1.1127x over previous
import functools

import jax
import jax.numpy as jnp
from jax import lax
from jax.experimental import pallas as pl
from jax.experimental.pallas import tpu as pltpu
from jax.experimental.pallas import tpu_sc as plsc

B, N, D = 4, 4096, 1024
NC, NS = 2, 16
NW = NC * NS
NPW = N // NW
C = 16
NCH = NPW // C
TOT = NCH * B
CW = C * D
NSLOT = 5
P = 3

_mesh = plsc.VectorSubcoreMesh(core_axis_name="c", subcore_axis_name="s")


@functools.partial(
    pl.kernel,
    mesh=_mesh,
    out_type=jax.ShapeDtypeStruct((B * N, D), jnp.float32),
    scratch_types=(
        [pltpu.VMEM((C, D), jnp.float32)] * 2
        + [pltpu.VMEM((C, D), jnp.float32)] * NSLOT
        + [pltpu.SemaphoreType.DMA] * 2
        + [pltpu.SemaphoreType.DMA] * NSLOT
        + [pltpu.SemaphoreType.DMA] * NSLOT
    ),
)
def _pos_add(x_hbm, t_hbm, o_hbm, *rest):
    tbufs = rest[:2]
    xbufs = rest[2:2 + NSLOT]
    tsems = rest[2 + NSLOT:4 + NSLOT]
    ldsems = rest[4 + NSLOT:4 + 2 * NSLOT]
    stsems = rest[4 + 2 * NSLOT:4 + 3 * NSLOT]

    wid = lax.axis_index("s") * NC + lax.axis_index("c")
    nbase = wid * NPW

    def x_slice(k):
        nc_, b_ = k // B, k % B
        return pl.ds(b_ * N + nbase + nc_ * C, C)

    def t_slice(nc_):
        return pl.ds(nbase + nc_ * C, C)

    t_h = [None, None]
    t_h[0] = pltpu.async_copy(t_hbm.at[t_slice(0)], tbufs[0], tsems[0])
    ld_h = [None] * NSLOT
    st_h = [None] * NSLOT
    for k in range(min(P, TOT)):
        ld_h[k % NSLOT] = pltpu.async_copy(
            x_hbm.at[x_slice(k)], xbufs[k % NSLOT], ldsems[k % NSLOT])

    tbuf = tbufs[0]
    for k in range(TOT):
        s = k % NSLOT
        nc_, b_ = k // B, k % B
        if b_ == 0:
            tbuf = tbufs[nc_ % 2]
            t_h[nc_ % 2].wait()
        if b_ == 1 and nc_ + 1 < NCH:
            nn = nc_ + 1
            t_h[nn % 2] = pltpu.async_copy(
                t_hbm.at[t_slice(nn)], tbufs[nn % 2], tsems[nn % 2])
        ld_h[s].wait()
        xb = xbufs[s]

        @plsc.parallel_loop(0, CW, step=16, unroll=8)
        def add_body(i, xb=xb, tbuf=tbuf):
            r = i >> 10
            c = pl.multiple_of(i & (D - 1), 16)
            sl = pl.ds(c, 16)
            plsc.addupdate(xb.at[r, sl], tbuf[r, sl])

        st_h[s] = pltpu.async_copy(xb, o_hbm.at[x_slice(k)], stsems[s])
        kn = k + P
        if kn < TOT:
            sn = kn % NSLOT
            if st_h[sn] is not None:
                st_h[sn].wait()
                st_h[sn] = None
            ld_h[sn] = pltpu.async_copy(x_hbm.at[x_slice(kn)], xbufs[sn], ldsems[sn])

    for h in st_h:
        if h is not None:
            h.wait()


def kernel(x, table):
    out = _pos_add(x.reshape(B * N, D), table)
    return out.reshape(x.shape)
